# baseline (speedup 1.0000x reference)
import functools

import jax
import jax.numpy as jnp
from jax.experimental import pallas as pl
from jax.experimental.pallas import tpu as pltpu

NUM_SEGMENTS = 1024
WIN = 128
NPAR = 2
_SELU_ALPHA = 1.6732632423543772
_SELU_SCALE = 1.0507009873554805


def _selu(x):
    return _SELU_SCALE * jnp.where(x > 0, x, _SELU_ALPHA * (jnp.exp(x) - 1.0))


def _onehot_update(seg, x_hi, x_lo, base, width):
    b = seg.shape[0]
    iota = jax.lax.broadcasted_iota(jnp.int32, (width, b), 0) + base
    onehot = (iota == seg[None, :]).astype(jnp.bfloat16)
    sums = (jax.lax.dot_general(
                onehot, x_hi, (((1,), (0,)), ((), ())),
                preferred_element_type=jnp.float32)
            + jax.lax.dot_general(
                onehot, x_lo, (((1,), (0,)), ((), ())),
                preferred_element_type=jnp.float32))
    cnts = jnp.sum(onehot.astype(jnp.float32), axis=1, keepdims=True)
    return sums, cnts


def _seg_sum_kernel(meta_ref, x_ref, seg_ref, sums_ref, cnts_ref,
                    acc_ref, cnt_ref, *, npb):
    p = pl.program_id(0)
    j = pl.program_id(1)
    g = p * npb + j

    @pl.when(j == 0)
    def _init():
        acc_ref[...] = jnp.zeros_like(acc_ref)
        cnt_ref[...] = jnp.zeros_like(cnt_ref)

    seg = seg_ref[0, 0, :]
    x = x_ref[...]
    x_hi = x.astype(jnp.bfloat16)
    x_lo = (x - x_hi.astype(jnp.float32)).astype(jnp.bfloat16)
    base = meta_ref[g, 0]
    narrow = meta_ref[g, 1] == 1

    @pl.when(narrow)
    def _narrow():
        sums, cnts = _onehot_update(seg, x_hi, x_lo, base, WIN)
        acc_ref[pl.ds(base, WIN), :] += sums
        cnt_ref[pl.ds(base, WIN), :] += cnts

    @pl.when(jnp.logical_not(narrow))
    def _wide():
        sums, cnts = _onehot_update(seg, x_hi, x_lo, 0, NUM_SEGMENTS)
        acc_ref[...] += sums
        cnt_ref[...] += cnts

    @pl.when(j == npb - 1)
    def _flush():
        sums_ref[0] = acc_ref[...]
        cnts_ref[0] = cnt_ref[...]


def _combine_mlp_kernel(sums_ref, cnts_ref, w1_ref, b1_ref, w2_ref, b2_ref,
                        out_ref):
    sums = jnp.sum(sums_ref[...], axis=0)
    cnts = jnp.sum(cnts_ref[...], axis=0)
    mean = sums / jnp.maximum(cnts, 1.0)
    h = _selu(jax.lax.dot_general(
        mean, w1_ref[...], (((1,), (0,)), ((), ())),
        preferred_element_type=jnp.float32,
        precision=jax.lax.Precision.HIGHEST) + b1_ref[...])
    out_ref[...] = jax.lax.dot_general(
        h, w2_ref[...], (((1,), (0,)), ((), ())),
        preferred_element_type=jnp.float32,
        precision=jax.lax.Precision.HIGHEST) + b2_ref[...]


def kernel(node_invariant_features, batch, W_pe, b_pe, W1, b1, W2, b2):
    x = node_invariant_features
    n, d = x.shape
    blk = 2000
    nblk = n // blk
    npb = nblk // NPAR
    assert npb * NPAR == nblk and nblk * blk == n
    seg = batch.astype(jnp.int32)
    seg3d = seg.reshape(nblk, 1, blk)
    starts = seg3d[:, 0, 0]
    ends = seg3d[:, 0, blk - 1]
    bases = jnp.minimum((starts // 8) * 8, NUM_SEGMENTS - WIN)
    narrow = (ends - bases) < WIN
    meta = jnp.stack([bases, narrow.astype(jnp.int32)], axis=1)

    sums, cnts = pl.pallas_call(
        functools.partial(_seg_sum_kernel, npb=npb),
        grid=(NPAR, npb),
        in_specs=[
            pl.BlockSpec(memory_space=pltpu.SMEM),
            pl.BlockSpec((blk, d), lambda p, j: (p * npb + j, 0)),
            pl.BlockSpec((1, 1, blk), lambda p, j: (p * npb + j, 0, 0)),
        ],
        out_specs=[
            pl.BlockSpec((1, NUM_SEGMENTS, d), lambda p, j: (p, 0, 0)),
            pl.BlockSpec((1, NUM_SEGMENTS, 1), lambda p, j: (p, 0, 0)),
        ],
        out_shape=[
            jax.ShapeDtypeStruct((NPAR, NUM_SEGMENTS, d), jnp.float32),
            jax.ShapeDtypeStruct((NPAR, NUM_SEGMENTS, 1), jnp.float32),
        ],
        scratch_shapes=[
            pltpu.VMEM((NUM_SEGMENTS, d), jnp.float32),
            pltpu.VMEM((NUM_SEGMENTS, 1), jnp.float32),
        ],
        compiler_params=pltpu.CompilerParams(
            dimension_semantics=("parallel", "arbitrary")),
    )(meta, x, seg3d)

    b1r = b1.reshape(1, -1)
    b2r = b2.reshape(1, -1)
    out = pl.pallas_call(
        _combine_mlp_kernel,
        out_shape=jax.ShapeDtypeStruct((NUM_SEGMENTS, 1), jnp.float32),
    )(sums, cnts, W1, b1r, W2, b2r)
    return out


# speedup vs baseline: 1.0891x; 1.0891x over previous
import functools

import jax
import jax.numpy as jnp
from jax.experimental import pallas as pl
from jax.experimental.pallas import tpu as pltpu

NUM_SEGMENTS = 1024
WIN = 128
_SELU_ALPHA = 1.6732632423543772
_SELU_SCALE = 1.0507009873554805


def _selu(x):
    return _SELU_SCALE * jnp.where(x > 0, x, _SELU_ALPHA * (jnp.exp(x) - 1.0))


def _onehot_update(seg, x_hi, x_lo, base, width):
    b = seg.shape[0]
    iota = jax.lax.broadcasted_iota(jnp.int32, (width, b), 0) + base
    onehot = (iota == seg[None, :]).astype(jnp.bfloat16)
    sums = (jax.lax.dot_general(
                onehot, x_hi, (((1,), (0,)), ((), ())),
                preferred_element_type=jnp.float32)
            + jax.lax.dot_general(
                onehot, x_lo, (((1,), (0,)), ((), ())),
                preferred_element_type=jnp.float32))
    cnts = jnp.sum(onehot.astype(jnp.float32), axis=1, keepdims=True)
    return sums, cnts


def _seg_mlp_kernel(meta_ref, x_ref, seg_ref, w1_ref, b1_ref, w2_ref, b2_ref,
                    out_ref, acc_ref, cnt_ref, *, nblk):
    i = pl.program_id(0)

    @pl.when(i == 0)
    def _init():
        acc_ref[...] = jnp.zeros_like(acc_ref)
        cnt_ref[...] = jnp.zeros_like(cnt_ref)

    seg = seg_ref[0, 0, :]
    x = x_ref[...]
    x_hi = x.astype(jnp.bfloat16)
    x_lo = (x - x_hi.astype(jnp.float32)).astype(jnp.bfloat16)
    base = meta_ref[i, 0]
    narrow = meta_ref[i, 1] == 1

    @pl.when(narrow)
    def _narrow():
        sums, cnts = _onehot_update(seg, x_hi, x_lo, base, WIN)
        acc_ref[pl.ds(base, WIN), :] += sums
        cnt_ref[pl.ds(base, WIN), :] += cnts

    @pl.when(jnp.logical_not(narrow))
    def _wide():
        sums, cnts = _onehot_update(seg, x_hi, x_lo, 0, NUM_SEGMENTS)
        acc_ref[...] += sums
        cnt_ref[...] += cnts

    @pl.when(i == nblk - 1)
    def _epilogue():
        mean = acc_ref[...] / jnp.maximum(cnt_ref[...], 1.0)
        h = _selu(jax.lax.dot_general(
            mean, w1_ref[...], (((1,), (0,)), ((), ())),
            preferred_element_type=jnp.float32,
            precision=jax.lax.Precision.HIGHEST) + b1_ref[...])
        out_ref[...] = jax.lax.dot_general(
            h, w2_ref[...], (((1,), (0,)), ((), ())),
            preferred_element_type=jnp.float32,
            precision=jax.lax.Precision.HIGHEST) + b2_ref[...]


def kernel(node_invariant_features, batch, W_pe, b_pe, W1, b1, W2, b2):
    x = node_invariant_features
    n, d = x.shape
    blk = 2000
    nblk = n // blk
    assert nblk * blk == n
    seg = batch.astype(jnp.int32)
    seg3d = seg.reshape(nblk, 1, blk)
    starts = seg3d[:, 0, 0]
    ends = seg3d[:, 0, blk - 1]
    bases = jnp.minimum((starts // 8) * 8, NUM_SEGMENTS - WIN)
    narrow = (ends - bases) < WIN
    meta = jnp.stack([bases, narrow.astype(jnp.int32)], axis=1)
    b1r = b1.reshape(1, -1)
    b2r = b2.reshape(1, -1)

    out = pl.pallas_call(
        functools.partial(_seg_mlp_kernel, nblk=nblk),
        grid=(nblk,),
        in_specs=[
            pl.BlockSpec(memory_space=pltpu.SMEM),
            pl.BlockSpec((blk, d), lambda i: (i, 0)),
            pl.BlockSpec((1, 1, blk), lambda i: (i, 0, 0)),
            pl.BlockSpec(W1.shape, lambda i: (0, 0)),
            pl.BlockSpec(b1r.shape, lambda i: (0, 0)),
            pl.BlockSpec(W2.shape, lambda i: (0, 0)),
            pl.BlockSpec(b2r.shape, lambda i: (0, 0)),
        ],
        out_specs=pl.BlockSpec((NUM_SEGMENTS, 1), lambda i: (0, 0)),
        out_shape=jax.ShapeDtypeStruct((NUM_SEGMENTS, 1), jnp.float32),
        scratch_shapes=[
            pltpu.VMEM((NUM_SEGMENTS, d), jnp.float32),
            pltpu.VMEM((NUM_SEGMENTS, 1), jnp.float32),
        ],
        compiler_params=pltpu.CompilerParams(
            dimension_semantics=("arbitrary",)),
    )(meta, x, seg3d, W1, b1r, W2, b2r)
    return out


# speedup vs baseline: 1.2521x; 1.1497x over previous
import functools

import jax
import jax.numpy as jnp
from jax.experimental import pallas as pl
from jax.experimental.pallas import tpu as pltpu

NUM_SEGMENTS = 1024
WIN = 64
_SELU_ALPHA = 1.6732632423543772
_SELU_SCALE = 1.0507009873554805


def _selu(x):
    return _SELU_SCALE * jnp.where(x > 0, x, _SELU_ALPHA * (jnp.exp(x) - 1.0))


def _onehot_update(segr, x_hi, iota):
    onehot = (iota == segr[None, :]).astype(jnp.bfloat16)
    sums = jax.lax.dot_general(
        onehot, x_hi, (((1,), (0,)), ((), ())),
        preferred_element_type=jnp.float32)
    cnts = jnp.sum(onehot, axis=1, keepdims=True, dtype=jnp.float32)
    return sums, cnts


def _seg_mlp_kernel(meta_ref, x_ref, seg_ref, w1_ref, b1_ref, w2_ref, b2_ref,
                    out_ref, acc_ref, cnt_ref, iota_ref, *, nblk):
    i = pl.program_id(0)
    b = seg_ref.shape[-1]

    @pl.when(i == 0)
    def _init():
        acc_ref[...] = jnp.zeros_like(acc_ref)
        cnt_ref[...] = jnp.zeros_like(cnt_ref)
        iota_ref[...] = jax.lax.broadcasted_iota(jnp.int32, (WIN, b), 0)

    seg = seg_ref[0, 0, :]
    x = x_ref[...]
    x_hi = x.astype(jnp.bfloat16)
    base = meta_ref[i, 0]
    narrow = meta_ref[i, 1] == 1

    @pl.when(narrow)
    def _narrow():
        sums, cnts = _onehot_update(seg - base, x_hi, iota_ref[...])
        acc_ref[pl.ds(base, WIN), :] += sums
        cnt_ref[pl.ds(base, WIN), :] += cnts

    @pl.when(jnp.logical_not(narrow))
    def _wide():
        iota = jax.lax.broadcasted_iota(jnp.int32, (NUM_SEGMENTS, b), 0)
        sums, cnts = _onehot_update(seg, x_hi, iota)
        acc_ref[...] += sums
        cnt_ref[...] += cnts

    @pl.when(i == nblk - 1)
    def _epilogue():
        mean = acc_ref[...] / jnp.maximum(cnt_ref[...], 1.0)
        h = _selu(jax.lax.dot_general(
            mean, w1_ref[...], (((1,), (0,)), ((), ())),
            preferred_element_type=jnp.float32,
            precision=jax.lax.Precision.HIGHEST) + b1_ref[...])
        out_ref[...] = jax.lax.dot_general(
            h, w2_ref[...], (((1,), (0,)), ((), ())),
            preferred_element_type=jnp.float32,
            precision=jax.lax.Precision.HIGHEST) + b2_ref[...]


def kernel(node_invariant_features, batch, W_pe, b_pe, W1, b1, W2, b2):
    x = node_invariant_features
    n, d = x.shape
    blk = 2000
    nblk = n // blk
    assert nblk * blk == n
    seg = batch.astype(jnp.int32)
    seg3d = seg.reshape(nblk, 1, blk)
    starts = seg3d[:, 0, 0]
    ends = seg3d[:, 0, blk - 1]
    bases = jnp.minimum((starts // 8) * 8, NUM_SEGMENTS - WIN)
    narrow = (ends - bases) < WIN
    meta = jnp.stack([bases, narrow.astype(jnp.int32)], axis=1)
    b1r = b1.reshape(1, -1)
    b2r = b2.reshape(1, -1)

    out = pl.pallas_call(
        functools.partial(_seg_mlp_kernel, nblk=nblk),
        grid=(nblk,),
        in_specs=[
            pl.BlockSpec(memory_space=pltpu.SMEM),
            pl.BlockSpec((blk, d), lambda i: (i, 0)),
            pl.BlockSpec((1, 1, blk), lambda i: (i, 0, 0)),
            pl.BlockSpec(W1.shape, lambda i: (0, 0)),
            pl.BlockSpec(b1r.shape, lambda i: (0, 0)),
            pl.BlockSpec(W2.shape, lambda i: (0, 0)),
            pl.BlockSpec(b2r.shape, lambda i: (0, 0)),
        ],
        out_specs=pl.BlockSpec((NUM_SEGMENTS, 1), lambda i: (0, 0)),
        out_shape=jax.ShapeDtypeStruct((NUM_SEGMENTS, 1), jnp.float32),
        scratch_shapes=[
            pltpu.VMEM((NUM_SEGMENTS, d), jnp.float32),
            pltpu.VMEM((NUM_SEGMENTS, 1), jnp.float32),
            pltpu.VMEM((WIN, blk), jnp.int32),
        ],
        compiler_params=pltpu.CompilerParams(
            dimension_semantics=("arbitrary",)),
    )(meta, x, seg3d, W1, b1r, W2, b2r)
    return out


# speedup vs baseline: 1.7490x; 1.3968x over previous
import functools

import jax
import jax.numpy as jnp
from jax.experimental import pallas as pl
from jax.experimental.pallas import tpu as pltpu

NUM_SEGMENTS = 1024
WIN = 64
_SELU_ALPHA = 1.6732632423543772
_SELU_SCALE = 1.0507009873554805


def _selu(x):
    return _SELU_SCALE * jnp.where(x > 0, x, _SELU_ALPHA * (jnp.exp(x) - 1.0))


def _onehot_update(segr, x_hi, iota):
    onehot = (iota == segr[None, :]).astype(jnp.bfloat16)
    sums = jax.lax.dot_general(
        onehot, x_hi, (((1,), (0,)), ((), ())),
        preferred_element_type=jnp.float32)
    cnts = jnp.sum(onehot, axis=1, keepdims=True, dtype=jnp.float32)
    return sums, cnts


def _seg_mlp_kernel(meta_ref, x_ref, seg_ref, w1_ref, b1_ref, w2_ref, b2_ref,
                    out_ref, acc_ref, cnt_ref, iota_ref, *, nblk):
    i = pl.program_id(0)
    b = seg_ref.shape[-1]

    @pl.when(i == 0)
    def _init():
        acc_ref[...] = jnp.zeros_like(acc_ref)
        cnt_ref[...] = jnp.zeros_like(cnt_ref)
        iota_ref[...] = jax.lax.broadcasted_iota(jnp.int32, (WIN, b), 0)

    seg = seg_ref[0, 0, :]
    x = x_ref[...]
    x_hi = x.astype(jnp.bfloat16)
    base = meta_ref[i, 0]
    narrow = meta_ref[i, 1] == 1

    @pl.when(narrow)
    def _narrow():
        sums, cnts = _onehot_update(seg - base, x_hi, iota_ref[...])
        acc_ref[pl.ds(base, WIN), :] += sums
        cnt_ref[pl.ds(base, WIN), :] += cnts

    @pl.when(jnp.logical_not(narrow))
    def _wide():
        iota = jax.lax.broadcasted_iota(jnp.int32, (NUM_SEGMENTS, b), 0)
        sums, cnts = _onehot_update(seg, x_hi, iota)
        acc_ref[...] += sums
        cnt_ref[...] += cnts

    @pl.when(i == nblk - 1)
    def _epilogue():
        mean = acc_ref[...] / jnp.maximum(cnt_ref[...], 1.0)
        h = _selu(jax.lax.dot_general(
            mean, w1_ref[...], (((1,), (0,)), ((), ())),
            preferred_element_type=jnp.float32,
            precision=jax.lax.Precision.HIGHEST) + b1_ref[...])
        out_ref[...] = jax.lax.dot_general(
            h, w2_ref[...], (((1,), (0,)), ((), ())),
            preferred_element_type=jnp.float32,
            precision=jax.lax.Precision.HIGHEST) + b2_ref[...]


def kernel(node_invariant_features, batch, W_pe, b_pe, W1, b1, W2, b2):
    x = node_invariant_features
    n, d = x.shape
    blk = 4000
    nblk = n // blk
    assert nblk * blk == n
    seg = batch.astype(jnp.int32)
    seg3d = seg.reshape(nblk, 1, blk)
    starts = seg3d[:, 0, 0]
    ends = seg3d[:, 0, blk - 1]
    bases = jnp.minimum((starts // 8) * 8, NUM_SEGMENTS - WIN)
    narrow = (ends - bases) < WIN
    meta = jnp.stack([bases, narrow.astype(jnp.int32)], axis=1)
    b1r = b1.reshape(1, -1)
    b2r = b2.reshape(1, -1)

    out = pl.pallas_call(
        functools.partial(_seg_mlp_kernel, nblk=nblk),
        grid=(nblk,),
        in_specs=[
            pl.BlockSpec(memory_space=pltpu.SMEM),
            pl.BlockSpec((blk, d), lambda i: (i, 0)),
            pl.BlockSpec((1, 1, blk), lambda i: (i, 0, 0)),
            pl.BlockSpec(W1.shape, lambda i: (0, 0)),
            pl.BlockSpec(b1r.shape, lambda i: (0, 0)),
            pl.BlockSpec(W2.shape, lambda i: (0, 0)),
            pl.BlockSpec(b2r.shape, lambda i: (0, 0)),
        ],
        out_specs=pl.BlockSpec((NUM_SEGMENTS, 1), lambda i: (0, 0)),
        out_shape=jax.ShapeDtypeStruct((NUM_SEGMENTS, 1), jnp.float32),
        scratch_shapes=[
            pltpu.VMEM((NUM_SEGMENTS, d), jnp.float32),
            pltpu.VMEM((NUM_SEGMENTS, 1), jnp.float32),
            pltpu.VMEM((WIN, blk), jnp.int32),
        ],
        compiler_params=pltpu.CompilerParams(
            dimension_semantics=("arbitrary",)),
    )(meta, x, seg3d, W1, b1r, W2, b2r)
    return out


# speedup vs baseline: 1.9081x; 1.0910x over previous
import functools

import jax
import jax.numpy as jnp
from jax.experimental import pallas as pl
from jax.experimental.pallas import tpu as pltpu

NUM_SEGMENTS = 1024
WIN = 64
_SELU_ALPHA = 1.6732632423543772
_SELU_SCALE = 1.0507009873554805


def _selu(x):
    return _SELU_SCALE * jnp.where(x > 0, x, _SELU_ALPHA * (jnp.exp(x) - 1.0))


def _onehot_update(segr, x_hi, iota):
    onehot = (iota == segr[None, :]).astype(jnp.bfloat16)
    sums = jax.lax.dot_general(
        onehot, x_hi, (((1,), (0,)), ((), ())),
        preferred_element_type=jnp.float32)
    cnts = jnp.sum(onehot, axis=1, keepdims=True, dtype=jnp.float32)
    return sums, cnts


def _seg_mlp_kernel(meta_ref, x_ref, seg_ref, w1_ref, b1_ref, w2_ref, b2_ref,
                    out_ref, acc_ref, cnt_ref, iota_ref, *, nblk):
    i = pl.program_id(0)
    b = seg_ref.shape[-1]

    @pl.when(i == 0)
    def _init():
        acc_ref[...] = jnp.zeros_like(acc_ref)
        cnt_ref[...] = jnp.zeros_like(cnt_ref)
        iota_ref[...] = jax.lax.broadcasted_iota(jnp.int32, (WIN, b), 0)

    seg = seg_ref[0, 0, :]
    x = x_ref[...]
    x_hi = x.astype(jnp.bfloat16)
    base = meta_ref[i, 0]
    narrow = meta_ref[i, 1] == 1

    @pl.when(narrow)
    def _narrow():
        sums, cnts = _onehot_update(seg - base, x_hi, iota_ref[...])
        acc_ref[pl.ds(base, WIN), :] += sums
        cnt_ref[pl.ds(base, WIN), :] += cnts

    @pl.when(jnp.logical_not(narrow))
    def _wide():
        iota = jax.lax.broadcasted_iota(jnp.int32, (NUM_SEGMENTS, b), 0)
        sums, cnts = _onehot_update(seg, x_hi, iota)
        acc_ref[...] += sums
        cnt_ref[...] += cnts

    @pl.when(i == nblk - 1)
    def _epilogue():
        mean = acc_ref[...] / jnp.maximum(cnt_ref[...], 1.0)
        h = _selu(jax.lax.dot_general(
            mean, w1_ref[...], (((1,), (0,)), ((), ())),
            preferred_element_type=jnp.float32,
            precision=jax.lax.Precision.HIGHEST) + b1_ref[...])
        out_ref[...] = jax.lax.dot_general(
            h, w2_ref[...], (((1,), (0,)), ((), ())),
            preferred_element_type=jnp.float32,
            precision=jax.lax.Precision.HIGHEST) + b2_ref[...]


def kernel(node_invariant_features, batch, W_pe, b_pe, W1, b1, W2, b2):
    x = node_invariant_features
    n, d = x.shape
    blk = 5000
    nblk = n // blk
    assert nblk * blk == n
    seg = batch.astype(jnp.int32)
    seg3d = seg.reshape(nblk, 1, blk)
    starts = seg3d[:, 0, 0]
    ends = seg3d[:, 0, blk - 1]
    bases = jnp.minimum((starts // 8) * 8, NUM_SEGMENTS - WIN)
    narrow = (ends - bases) < WIN
    meta = jnp.stack([bases, narrow.astype(jnp.int32)], axis=1)
    b1r = b1.reshape(1, -1)
    b2r = b2.reshape(1, -1)

    out = pl.pallas_call(
        functools.partial(_seg_mlp_kernel, nblk=nblk),
        grid=(nblk,),
        in_specs=[
            pl.BlockSpec(memory_space=pltpu.SMEM),
            pl.BlockSpec((blk, d), lambda i: (i, 0)),
            pl.BlockSpec((1, 1, blk), lambda i: (i, 0, 0)),
            pl.BlockSpec(W1.shape, lambda i: (0, 0)),
            pl.BlockSpec(b1r.shape, lambda i: (0, 0)),
            pl.BlockSpec(W2.shape, lambda i: (0, 0)),
            pl.BlockSpec(b2r.shape, lambda i: (0, 0)),
        ],
        out_specs=pl.BlockSpec((NUM_SEGMENTS, 1), lambda i: (0, 0)),
        out_shape=jax.ShapeDtypeStruct((NUM_SEGMENTS, 1), jnp.float32),
        scratch_shapes=[
            pltpu.VMEM((NUM_SEGMENTS, d), jnp.float32),
            pltpu.VMEM((NUM_SEGMENTS, 1), jnp.float32),
            pltpu.VMEM((WIN, blk), jnp.int32),
        ],
        compiler_params=pltpu.CompilerParams(
            dimension_semantics=("arbitrary",)),
    )(meta, x, seg3d, W1, b1r, W2, b2r)
    return out
